# baseline (speedup 1.0000x reference)
import functools

import jax
import jax.numpy as jnp
from jax import lax
from jax.experimental import pallas as pl
from jax.experimental.pallas import tpu as pltpu
from jax.experimental.pallas import tpu_sc as plsc

D_TYPE_ = 128
VOCAB_ = 20
D_FEAT_ = D_TYPE_ + VOCAB_
D_PAD_ = 256

GATHER_WINDOW = 128


def _table2_body(table_ref, w1_ref, b1_ref, w2_ref, b2_ref, out_ref):
    row = lax.broadcasted_iota(jnp.int32, (VOCAB_, VOCAB_), 0)
    col = lax.broadcasted_iota(jnp.int32, (VOCAB_, VOCAB_), 1)
    eye = jnp.where(row == col, 1.0, 0.0).astype(jnp.float32)
    rows = jnp.concatenate([table_ref[...], eye], axis=1)
    h = jnp.dot(rows, w1_ref[...], preferred_element_type=jnp.float32)
    h = jnp.maximum(h + b1_ref[...], 0.0)
    out = jnp.dot(h, w2_ref[...], preferred_element_type=jnp.float32)
    out = out + b2_ref[...]
    out_ref[...] = jnp.pad(out, ((0, 0), (0, D_PAD_ - D_FEAT_)))


def _compute_table2(table, W1, b1, W2, b2):
    return pl.pallas_call(
        _table2_body,
        out_shape=jax.ShapeDtypeStruct((VOCAB_, D_PAD_), jnp.float32),
    )(table, W1, b1.reshape(1, D_FEAT_), W2, b2.reshape(1, D_FEAT_))


def _sc_gather_v3(table2, x):
    B, L = x.shape
    info = plsc.get_sparse_core_info()
    nw = info.num_cores * info.num_subcores
    rw = B // nw
    assert B % nw == 0 and rw % 2 == 0

    chunk_offs = list(range(0, D_FEAT_ - 15, 16))
    if chunk_offs[-1] + 16 < D_FEAT_:
        chunk_offs.append(D_FEAT_ - 16)
    tok_groups = [g * 16 for g in range(L // 16)]
    if L % 16:
        tok_groups.append(L - 16)

    mesh = plsc.VectorSubcoreMesh(core_axis_name="core",
                                  subcore_axis_name="subcore")

    @functools.partial(
        pl.kernel,
        out_type=jax.ShapeDtypeStruct((B, L, D_FEAT_), jnp.float32),
        mesh=mesh,
        scratch_types=[
            pltpu.VMEM((VOCAB_, D_FEAT_), jnp.float32),
            pltpu.VMEM((16, L), jnp.int32),
            pltpu.VMEM((2, L, D_FEAT_), jnp.float32),
            pltpu.SemaphoreType.DMA,
            pltpu.SemaphoreType.DMA,
        ],
    )
    def kern(t2_hbm, x_hbm, o_hbm, t2_vmem, ibuf, obuf, sem0, sem1):
        wid = (lax.axis_index("subcore") * info.num_cores
               + lax.axis_index("core"))
        row0 = wid * rw
        sems = (sem0, sem1)
        pltpu.sync_copy(t2_hbm, t2_vmem)

        def compute(s_local, phase):
            ob = obuf.at[phase]

            def do_group(c0):
                vec = ibuf[s_local, pl.ds(c0, 16)]
                for k in range(16):
                    xr = vec[k]
                    tok = c0 + k
                    for c in chunk_offs:
                        ob[tok, pl.ds(c, 16)] = t2_vmem[xr, pl.ds(c, 16)]

            @plsc.parallel_loop(0, L // 16, unroll=2)
            def _(g):
                do_group(g * 16)

            if L % 16:
                do_group(L - 16)

        @pl.loop(0, rw // 16)
        def _(blk):
            pltpu.sync_copy(x_hbm.at[pl.ds(row0 + blk * 16, 16)], ibuf)

            @pl.loop(0, 16)
            def _(s_local):
                s = blk * 16 + s_local
                compute(s_local, 0)
                pltpu.sync_copy(obuf.at[0], o_hbm.at[row0 + s])

    return kern(table2, x)


SC_CHUNK = 256


def _sc_gather_v2(table2, idx_flat):
    num_indices = idx_flat.shape[0]
    info = plsc.get_sparse_core_info()
    nw = info.num_cores * info.num_subcores
    per_w = num_indices // nw
    steps = per_w // SC_CHUNK
    assert per_w % SC_CHUNK == 0

    chunk_offs = list(range(0, D_FEAT_ - 15, 16))
    if chunk_offs[-1] + 16 < D_FEAT_:
        chunk_offs.append(D_FEAT_ - 16)

    mesh = plsc.VectorSubcoreMesh(core_axis_name="core",
                                  subcore_axis_name="subcore")

    @functools.partial(
        pl.kernel,
        out_type=jax.ShapeDtypeStruct((num_indices, D_FEAT_), jnp.float32),
        mesh=mesh,
        scratch_types=[
            pltpu.VMEM((VOCAB_, D_FEAT_), jnp.float32),
            pltpu.VMEM((SC_CHUNK, D_FEAT_), jnp.float32),
            pltpu.VMEM((SC_CHUNK,), jnp.int32),
        ],
    )
    def kern(t2_hbm, i_hbm, o_hbm, t2_vmem, obuf, ibuf):
        wid = (lax.axis_index("subcore") * info.num_cores
               + lax.axis_index("core"))
        base = wid * per_w
        pltpu.sync_copy(t2_hbm, t2_vmem)

        @pl.loop(0, steps)
        def _(s):
            tok0 = base + s * SC_CHUNK
            pltpu.sync_copy(i_hbm.at[pl.ds(tok0, SC_CHUNK)], ibuf)

            @pl.loop(0, SC_CHUNK // 16)
            def _(g):
                vec = ibuf[pl.ds(g * 16, 16)]
                for k in range(16):
                    xr = vec[k]
                    r = g * 16 + k
                    for c in chunk_offs:
                        obuf[r, pl.ds(c, 16)] = t2_vmem[xr, pl.ds(c, 16)]

            pltpu.sync_copy(obuf, o_hbm.at[pl.ds(tok0, SC_CHUNK)])

    return kern(table2, idx_flat)


def _sc_gather(table2, idx_flat):
    num_indices = idx_flat.shape[0]
    idx2d = idx_flat.reshape(1, num_indices)
    mesh = plsc.VectorSubcoreMesh(core_axis_name="core",
                                  subcore_axis_name="subcore")

    @functools.partial(
        pl.kernel,
        out_type=jax.ShapeDtypeStruct((num_indices, D_FEAT_), jnp.float32),
        mesh=mesh,
    )
    def kernel(t2_hbm, i_hbm, o_hbm):
        chunk_offs = list(range(0, D_FEAT_ - 15, 16))
        if chunk_offs[-1] + 16 < D_FEAT_:
            chunk_offs.append(D_FEAT_ - 16)

        def body(i_vmem, o_vmem):
            def inner(g_vmem):
                pltpu.sync_copy(t2_hbm.at[i_vmem.at[0]], g_vmem)

                @pl.loop(0, GATHER_WINDOW)
                def _(r):
                    for c in chunk_offs[:1]:
                        o_vmem[r, pl.ds(c, 16)] = g_vmem[r, pl.ds(c, 16)]

            pl.run_scoped(inner,
                          pltpu.VMEM((GATHER_WINDOW, D_PAD_), jnp.float32))

        pltpu.emit_pipeline(
            body,
            grid=(num_indices // GATHER_WINDOW,),
            in_specs=[pl.BlockSpec((1, GATHER_WINDOW),
                                   index_map=lambda i: (0, i))],
            out_specs=[pl.BlockSpec((GATHER_WINDOW, D_FEAT_),
                                    index_map=lambda i: (i, 0))],
            core_axis_name=("core", "subcore"),
            dimension_semantics=(pltpu.PARALLEL,),
        )(i_hbm, o_hbm)

    return kernel(table2, idx2d)


TC_BLOCK = 2048


def _tc_expand_body(x_ref, t2_ref, out_ref):
    onehot = jnp.where(x_ref[...] == lax.broadcasted_iota(
        jnp.int32, (TC_BLOCK, VOCAB_), 1), 1.0, 0.0).astype(jnp.float32)
    out_ref[...] = jnp.dot(onehot, t2_ref[...],
                           preferred_element_type=jnp.float32)


def _tc_expand(table2, idx_flat):
    n = idx_flat.shape[0]
    return pl.pallas_call(
        _tc_expand_body,
        grid=(n // TC_BLOCK,),
        in_specs=[
            pl.BlockSpec((TC_BLOCK, 1), lambda i: (i, 0)),
            pl.BlockSpec((VOCAB_, D_FEAT_), lambda i: (0, 0)),
        ],
        out_specs=pl.BlockSpec((TC_BLOCK, D_FEAT_), lambda i: (i, 0)),
        out_shape=jax.ShapeDtypeStruct((n, D_FEAT_), jnp.float32),
    )(idx_flat.reshape(n, 1), table2[:, :D_FEAT_])


def kernel(x, table, W1, b1, W2, b2):
    table2 = _compute_table2(table, W1, b1, W2, b2)
    return _sc_gather_v3(table2[:, :D_FEAT_], x.astype(jnp.int32))


# speedup vs baseline: 1.1074x; 1.1074x over previous
import functools

import jax
import jax.numpy as jnp
from jax import lax
from jax.experimental import pallas as pl
from jax.experimental.pallas import tpu as pltpu
from jax.experimental.pallas import tpu_sc as plsc

D_TYPE_ = 128
VOCAB_ = 20
D_FEAT_ = D_TYPE_ + VOCAB_
D_PAD_ = 256

GATHER_WINDOW = 128


def _table2_body(table_ref, w1_ref, b1_ref, w2_ref, b2_ref, out_ref):
    row = lax.broadcasted_iota(jnp.int32, (VOCAB_, VOCAB_), 0)
    col = lax.broadcasted_iota(jnp.int32, (VOCAB_, VOCAB_), 1)
    eye = jnp.where(row == col, 1.0, 0.0).astype(jnp.float32)
    rows = jnp.concatenate([table_ref[...], eye], axis=1)
    h = jnp.dot(rows, w1_ref[...], preferred_element_type=jnp.float32)
    h = jnp.maximum(h + b1_ref[...], 0.0)
    out = jnp.dot(h, w2_ref[...], preferred_element_type=jnp.float32)
    out = out + b2_ref[...]
    out_ref[...] = jnp.pad(out, ((0, 0), (0, D_PAD_ - D_FEAT_)))


def _compute_table2(table, W1, b1, W2, b2):
    return pl.pallas_call(
        _table2_body,
        out_shape=jax.ShapeDtypeStruct((VOCAB_, D_PAD_), jnp.float32),
    )(table, W1, b1.reshape(1, D_FEAT_), W2, b2.reshape(1, D_FEAT_))


def _sc_gather_v3(table2, x):
    B, L = x.shape
    info = plsc.get_sparse_core_info()
    nw = info.num_cores * info.num_subcores
    rw = B // nw
    assert B % nw == 0 and rw % 2 == 0

    chunk_offs = list(range(0, D_FEAT_ - 15, 16))
    if chunk_offs[-1] + 16 < D_FEAT_:
        chunk_offs.append(D_FEAT_ - 16)
    tok_groups = [g * 16 for g in range(L // 16)]
    if L % 16:
        tok_groups.append(L - 16)

    mesh = plsc.VectorSubcoreMesh(core_axis_name="core",
                                  subcore_axis_name="subcore")

    @functools.partial(
        pl.kernel,
        out_type=jax.ShapeDtypeStruct((B * L, D_FEAT_), jnp.float32),
        mesh=mesh,
        scratch_types=[
            pltpu.VMEM((VOCAB_, D_FEAT_), jnp.float32),
            pltpu.VMEM((16, L), jnp.int32),
            pltpu.VMEM((2, L, D_FEAT_), jnp.float32),
            pltpu.SemaphoreType.DMA,
            pltpu.SemaphoreType.DMA,
        ],
    )
    def kern(t2_hbm, x_hbm, o_hbm, t2_vmem, ibuf, obuf, sem0, sem1):
        wid = (lax.axis_index("subcore") * info.num_cores
               + lax.axis_index("core"))
        row0 = wid * rw
        sems = (sem0, sem1)
        pltpu.sync_copy(t2_hbm, t2_vmem)

        def compute(s_local, phase):
            ob = obuf.at[phase]

            def do_group(c0):
                vec = ibuf[s_local, pl.ds(c0, 16)]
                for k in range(16):
                    xr = vec[k]
                    tok = c0 + k
                    for c in chunk_offs:
                        ob[tok, pl.ds(c, 16)] = t2_vmem[xr, pl.ds(c, 16)]

            @plsc.parallel_loop(0, L // 16, unroll=2)
            def _(g):
                do_group(g * 16)

            if L % 16:
                do_group(L - 16)

        @pl.loop(0, rw // 16)
        def _(blk):
            pltpu.sync_copy(x_hbm.at[pl.ds(row0 + blk * 16, 16)], ibuf)

            @pl.loop(0, 16)
            def _(s_local):
                s = blk * 16 + s_local
                compute(s_local, 0)
                pltpu.sync_copy(obuf.at[0],
                                o_hbm.at[pl.ds((row0 + s) * L, L)])

    return kern(table2, x)


SC_CHUNK = 256


def _sc_gather_v2(table2, idx_flat):
    num_indices = idx_flat.shape[0]
    info = plsc.get_sparse_core_info()
    nw = info.num_cores * info.num_subcores
    per_w = num_indices // nw
    steps = per_w // SC_CHUNK
    assert per_w % SC_CHUNK == 0

    chunk_offs = list(range(0, D_FEAT_ - 15, 16))
    if chunk_offs[-1] + 16 < D_FEAT_:
        chunk_offs.append(D_FEAT_ - 16)

    mesh = plsc.VectorSubcoreMesh(core_axis_name="core",
                                  subcore_axis_name="subcore")

    @functools.partial(
        pl.kernel,
        out_type=jax.ShapeDtypeStruct((num_indices, D_FEAT_), jnp.float32),
        mesh=mesh,
        scratch_types=[
            pltpu.VMEM((VOCAB_, D_FEAT_), jnp.float32),
            pltpu.VMEM((SC_CHUNK, D_FEAT_), jnp.float32),
            pltpu.VMEM((SC_CHUNK,), jnp.int32),
        ],
    )
    def kern(t2_hbm, i_hbm, o_hbm, t2_vmem, obuf, ibuf):
        wid = (lax.axis_index("subcore") * info.num_cores
               + lax.axis_index("core"))
        base = wid * per_w
        pltpu.sync_copy(t2_hbm, t2_vmem)

        @pl.loop(0, steps)
        def _(s):
            tok0 = base + s * SC_CHUNK
            pltpu.sync_copy(i_hbm.at[pl.ds(tok0, SC_CHUNK)], ibuf)

            @pl.loop(0, SC_CHUNK // 16)
            def _(g):
                vec = ibuf[pl.ds(g * 16, 16)]
                for k in range(16):
                    xr = vec[k]
                    r = g * 16 + k
                    for c in chunk_offs:
                        obuf[r, pl.ds(c, 16)] = t2_vmem[xr, pl.ds(c, 16)]

            pltpu.sync_copy(obuf, o_hbm.at[pl.ds(tok0, SC_CHUNK)])

    return kern(table2, idx_flat)


def _sc_gather(table2, idx_flat):
    num_indices = idx_flat.shape[0]
    idx2d = idx_flat.reshape(1, num_indices)
    mesh = plsc.VectorSubcoreMesh(core_axis_name="core",
                                  subcore_axis_name="subcore")

    @functools.partial(
        pl.kernel,
        out_type=jax.ShapeDtypeStruct((num_indices, D_FEAT_), jnp.float32),
        mesh=mesh,
    )
    def kernel(t2_hbm, i_hbm, o_hbm):
        chunk_offs = list(range(0, D_FEAT_ - 15, 16))
        if chunk_offs[-1] + 16 < D_FEAT_:
            chunk_offs.append(D_FEAT_ - 16)

        def body(i_vmem, o_vmem):
            def inner(g_vmem):
                pltpu.sync_copy(t2_hbm.at[i_vmem.at[0]], g_vmem)

                @pl.loop(0, GATHER_WINDOW)
                def _(r):
                    for c in chunk_offs[:1]:
                        o_vmem[r, pl.ds(c, 16)] = g_vmem[r, pl.ds(c, 16)]

            pl.run_scoped(inner,
                          pltpu.VMEM((GATHER_WINDOW, D_PAD_), jnp.float32))

        pltpu.emit_pipeline(
            body,
            grid=(num_indices // GATHER_WINDOW,),
            in_specs=[pl.BlockSpec((1, GATHER_WINDOW),
                                   index_map=lambda i: (0, i))],
            out_specs=[pl.BlockSpec((GATHER_WINDOW, D_FEAT_),
                                    index_map=lambda i: (i, 0))],
            core_axis_name=("core", "subcore"),
            dimension_semantics=(pltpu.PARALLEL,),
        )(i_hbm, o_hbm)

    return kernel(table2, idx2d)


TC_BLOCK = 2048


def _tc_expand_body(x_ref, t2_ref, out_ref):
    onehot = jnp.where(x_ref[...] == lax.broadcasted_iota(
        jnp.int32, (TC_BLOCK, VOCAB_), 1), 1.0, 0.0).astype(jnp.float32)
    out_ref[...] = jnp.dot(onehot, t2_ref[...],
                           preferred_element_type=jnp.float32)


def _tc_expand(table2, idx_flat):
    n = idx_flat.shape[0]
    return pl.pallas_call(
        _tc_expand_body,
        grid=(n // TC_BLOCK,),
        in_specs=[
            pl.BlockSpec((TC_BLOCK, 1), lambda i: (i, 0)),
            pl.BlockSpec((VOCAB_, D_FEAT_), lambda i: (0, 0)),
        ],
        out_specs=pl.BlockSpec((TC_BLOCK, D_FEAT_), lambda i: (i, 0)),
        out_shape=jax.ShapeDtypeStruct((n, D_FEAT_), jnp.float32),
    )(idx_flat.reshape(n, 1), table2[:, :D_FEAT_])


def kernel(x, table, W1, b1, W2, b2):
    B, L = x.shape
    table2 = _compute_table2(table, W1, b1, W2, b2)
    out = _sc_gather_v3(table2[:, :D_FEAT_], x.astype(jnp.int32))
    return out.reshape(B, L, D_FEAT_)


# speedup vs baseline: 1.2028x; 1.0861x over previous
import functools

import jax
import jax.numpy as jnp
from jax import lax
from jax.experimental import pallas as pl
from jax.experimental.pallas import tpu as pltpu
from jax.experimental.pallas import tpu_sc as plsc

D_TYPE_ = 128
VOCAB_ = 20
D_FEAT_ = D_TYPE_ + VOCAB_
D_PAD_ = 256

GATHER_WINDOW = 128


def _table2_body(table_ref, w1_ref, b1_ref, w2_ref, b2_ref, out_ref):
    row = lax.broadcasted_iota(jnp.int32, (VOCAB_, VOCAB_), 0)
    col = lax.broadcasted_iota(jnp.int32, (VOCAB_, VOCAB_), 1)
    eye = jnp.where(row == col, 1.0, 0.0).astype(jnp.float32)
    rows = jnp.concatenate([table_ref[...], eye], axis=1)
    h = jnp.dot(rows, w1_ref[...], preferred_element_type=jnp.float32)
    h = jnp.maximum(h + b1_ref[...], 0.0)
    out = jnp.dot(h, w2_ref[...], preferred_element_type=jnp.float32)
    out = out + b2_ref[...]
    out_ref[...] = jnp.pad(out, ((0, 0), (0, D_PAD_ - D_FEAT_)))


def _compute_table2(table, W1, b1, W2, b2):
    return pl.pallas_call(
        _table2_body,
        out_shape=jax.ShapeDtypeStruct((VOCAB_, D_PAD_), jnp.float32),
    )(table, W1, b1.reshape(1, D_FEAT_), W2, b2.reshape(1, D_FEAT_))


def _sc_gather_v3(table2, x):
    B, L = x.shape
    info = plsc.get_sparse_core_info()
    nw = info.num_cores * info.num_subcores
    rw = B // nw
    assert B % nw == 0 and rw % 2 == 0

    chunk_offs = list(range(0, D_FEAT_ - 15, 16))
    if chunk_offs[-1] + 16 < D_FEAT_:
        chunk_offs.append(D_FEAT_ - 16)
    tok_groups = [g * 16 for g in range(L // 16)]
    if L % 16:
        tok_groups.append(L - 16)

    mesh = plsc.VectorSubcoreMesh(core_axis_name="core",
                                  subcore_axis_name="subcore")

    @functools.partial(
        pl.kernel,
        out_type=jax.ShapeDtypeStruct((B * L, D_FEAT_), jnp.float32),
        mesh=mesh,
        scratch_types=[
            pltpu.VMEM((VOCAB_, D_FEAT_), jnp.float32),
            pltpu.VMEM((16, L), jnp.int32),
            pltpu.VMEM((2, L, D_FEAT_), jnp.float32),
            pltpu.SemaphoreType.DMA,
            pltpu.SemaphoreType.DMA,
        ],
    )
    def kern(t2_hbm, x_hbm, o_hbm, t2_vmem, ibuf, obuf, sem0, sem1):
        wid = (lax.axis_index("subcore") * info.num_cores
               + lax.axis_index("core"))
        row0 = wid * rw
        sems = (sem0, sem1)
        pltpu.sync_copy(t2_hbm, t2_vmem)

        def compute(s_local, phase):
            ob = obuf.at[phase]

            def do_group(c0):
                vec = ibuf[s_local, pl.ds(c0, 16)]
                for k in range(16):
                    xr = vec[k]
                    tok = c0 + k
                    for c in chunk_offs:
                        ob[tok, pl.ds(c, 16)] = t2_vmem[xr, pl.ds(c, 16)]

            @plsc.parallel_loop(0, L // 16, unroll=2)
            def _(g):
                do_group(g * 16)

            if L % 16:
                do_group(L - 16)

        @pl.loop(0, rw // 16)
        def _(blk):
            pltpu.sync_copy(x_hbm.at[pl.ds(row0 + blk * 16, 16)], ibuf)

            @pl.loop(0, 8)
            def _(it):
                for phase in range(2):
                    s_local = it * 2 + phase
                    s = blk * 16 + s_local

                    @pl.when(s >= 2)
                    def _():
                        pltpu.make_async_copy(
                            obuf.at[phase],
                            o_hbm.at[pl.ds((row0 + s) * L, L)],
                            sems[phase]).wait()

                    compute(s_local, phase)
                    pltpu.async_copy(obuf.at[phase],
                                     o_hbm.at[pl.ds((row0 + s) * L, L)],
                                     sems[phase])

        pltpu.make_async_copy(obuf.at[0],
                              o_hbm.at[pl.ds((row0 + rw - 2) * L, L)],
                              sem0).wait()
        pltpu.make_async_copy(obuf.at[1],
                              o_hbm.at[pl.ds((row0 + rw - 1) * L, L)],
                              sem1).wait()

    return kern(table2, x)


SC_CHUNK = 256


def _sc_gather_v2(table2, idx_flat):
    num_indices = idx_flat.shape[0]
    info = plsc.get_sparse_core_info()
    nw = info.num_cores * info.num_subcores
    per_w = num_indices // nw
    steps = per_w // SC_CHUNK
    assert per_w % SC_CHUNK == 0

    chunk_offs = list(range(0, D_FEAT_ - 15, 16))
    if chunk_offs[-1] + 16 < D_FEAT_:
        chunk_offs.append(D_FEAT_ - 16)

    mesh = plsc.VectorSubcoreMesh(core_axis_name="core",
                                  subcore_axis_name="subcore")

    @functools.partial(
        pl.kernel,
        out_type=jax.ShapeDtypeStruct((num_indices, D_FEAT_), jnp.float32),
        mesh=mesh,
        scratch_types=[
            pltpu.VMEM((VOCAB_, D_FEAT_), jnp.float32),
            pltpu.VMEM((SC_CHUNK, D_FEAT_), jnp.float32),
            pltpu.VMEM((SC_CHUNK,), jnp.int32),
        ],
    )
    def kern(t2_hbm, i_hbm, o_hbm, t2_vmem, obuf, ibuf):
        wid = (lax.axis_index("subcore") * info.num_cores
               + lax.axis_index("core"))
        base = wid * per_w
        pltpu.sync_copy(t2_hbm, t2_vmem)

        @pl.loop(0, steps)
        def _(s):
            tok0 = base + s * SC_CHUNK
            pltpu.sync_copy(i_hbm.at[pl.ds(tok0, SC_CHUNK)], ibuf)

            @pl.loop(0, SC_CHUNK // 16)
            def _(g):
                vec = ibuf[pl.ds(g * 16, 16)]
                for k in range(16):
                    xr = vec[k]
                    r = g * 16 + k
                    for c in chunk_offs:
                        obuf[r, pl.ds(c, 16)] = t2_vmem[xr, pl.ds(c, 16)]

            pltpu.sync_copy(obuf, o_hbm.at[pl.ds(tok0, SC_CHUNK)])

    return kern(table2, idx_flat)


def _sc_gather(table2, idx_flat):
    num_indices = idx_flat.shape[0]
    idx2d = idx_flat.reshape(1, num_indices)
    mesh = plsc.VectorSubcoreMesh(core_axis_name="core",
                                  subcore_axis_name="subcore")

    @functools.partial(
        pl.kernel,
        out_type=jax.ShapeDtypeStruct((num_indices, D_FEAT_), jnp.float32),
        mesh=mesh,
    )
    def kernel(t2_hbm, i_hbm, o_hbm):
        chunk_offs = list(range(0, D_FEAT_ - 15, 16))
        if chunk_offs[-1] + 16 < D_FEAT_:
            chunk_offs.append(D_FEAT_ - 16)

        def body(i_vmem, o_vmem):
            def inner(g_vmem):
                pltpu.sync_copy(t2_hbm.at[i_vmem.at[0]], g_vmem)

                @pl.loop(0, GATHER_WINDOW)
                def _(r):
                    for c in chunk_offs[:1]:
                        o_vmem[r, pl.ds(c, 16)] = g_vmem[r, pl.ds(c, 16)]

            pl.run_scoped(inner,
                          pltpu.VMEM((GATHER_WINDOW, D_PAD_), jnp.float32))

        pltpu.emit_pipeline(
            body,
            grid=(num_indices // GATHER_WINDOW,),
            in_specs=[pl.BlockSpec((1, GATHER_WINDOW),
                                   index_map=lambda i: (0, i))],
            out_specs=[pl.BlockSpec((GATHER_WINDOW, D_FEAT_),
                                    index_map=lambda i: (i, 0))],
            core_axis_name=("core", "subcore"),
            dimension_semantics=(pltpu.PARALLEL,),
        )(i_hbm, o_hbm)

    return kernel(table2, idx2d)


TC_BLOCK = 2048


def _tc_expand_body(x_ref, t2_ref, out_ref):
    onehot = jnp.where(x_ref[...] == lax.broadcasted_iota(
        jnp.int32, (TC_BLOCK, VOCAB_), 1), 1.0, 0.0).astype(jnp.float32)
    out_ref[...] = jnp.dot(onehot, t2_ref[...],
                           preferred_element_type=jnp.float32)


def _tc_expand(table2, idx_flat):
    n = idx_flat.shape[0]
    return pl.pallas_call(
        _tc_expand_body,
        grid=(n // TC_BLOCK,),
        in_specs=[
            pl.BlockSpec((TC_BLOCK, 1), lambda i: (i, 0)),
            pl.BlockSpec((VOCAB_, D_FEAT_), lambda i: (0, 0)),
        ],
        out_specs=pl.BlockSpec((TC_BLOCK, D_FEAT_), lambda i: (i, 0)),
        out_shape=jax.ShapeDtypeStruct((n, D_FEAT_), jnp.float32),
    )(idx_flat.reshape(n, 1), table2[:, :D_FEAT_])


def kernel(x, table, W1, b1, W2, b2):
    B, L = x.shape
    table2 = _compute_table2(table, W1, b1, W2, b2)
    out = _sc_gather_v3(table2[:, :D_FEAT_], x.astype(jnp.int32))
    return out.reshape(B, L, D_FEAT_)


# speedup vs baseline: 1.2070x; 1.0035x over previous
import functools

import jax
import jax.numpy as jnp
from jax import lax
from jax.experimental import pallas as pl
from jax.experimental.pallas import tpu as pltpu
from jax.experimental.pallas import tpu_sc as plsc

D_TYPE_ = 128
VOCAB_ = 20
D_FEAT_ = D_TYPE_ + VOCAB_


def _table2_body(table_ref, w1_ref, b1_ref, w2_ref, b2_ref, out_ref):
    row = lax.broadcasted_iota(jnp.int32, (VOCAB_, VOCAB_), 0)
    col = lax.broadcasted_iota(jnp.int32, (VOCAB_, VOCAB_), 1)
    eye = jnp.where(row == col, 1.0, 0.0).astype(jnp.float32)
    rows = jnp.concatenate([table_ref[...], eye], axis=1)
    h = jnp.dot(rows, w1_ref[...], preferred_element_type=jnp.float32)
    h = jnp.maximum(h + b1_ref[...], 0.0)
    out = jnp.dot(h, w2_ref[...], preferred_element_type=jnp.float32)
    out_ref[...] = out + b2_ref[...]


def _compute_table2(table, W1, b1, W2, b2):
    return pl.pallas_call(
        _table2_body,
        out_shape=jax.ShapeDtypeStruct((VOCAB_, D_FEAT_), jnp.float32),
    )(table, W1, b1.reshape(1, D_FEAT_), W2, b2.reshape(1, D_FEAT_))


def _sc_gather(table2, x):
    B, L = x.shape
    info = plsc.get_sparse_core_info()
    nw = info.num_cores * info.num_subcores
    rw = B // nw
    assert B % nw == 0 and rw % 16 == 0

    chunk_offs = list(range(0, D_FEAT_ - 15, 16))
    if chunk_offs[-1] + 16 < D_FEAT_:
        chunk_offs.append(D_FEAT_ - 16)

    mesh = plsc.VectorSubcoreMesh(core_axis_name="core",
                                  subcore_axis_name="subcore")

    @functools.partial(
        pl.kernel,
        out_type=jax.ShapeDtypeStruct((B * L, D_FEAT_), jnp.float32),
        mesh=mesh,
        scratch_types=[
            pltpu.VMEM((VOCAB_, D_FEAT_), jnp.float32),
            pltpu.VMEM((16, L), jnp.int32),
            pltpu.VMEM((2, L, D_FEAT_), jnp.float32),
            pltpu.SemaphoreType.DMA,
            pltpu.SemaphoreType.DMA,
        ],
    )
    def kern(t2_hbm, x_hbm, o_hbm, t2_vmem, ibuf, obuf, sem0, sem1):
        wid = (lax.axis_index("subcore") * info.num_cores
               + lax.axis_index("core"))
        row0 = wid * rw
        sems = (sem0, sem1)
        pltpu.sync_copy(t2_hbm, t2_vmem)

        def compute(s_local, phase):
            ob = obuf.at[phase]

            def do_group(c0):
                vec = ibuf[s_local, pl.ds(c0, 16)]
                for k in range(16):
                    xr = vec[k]
                    tok = c0 + k
                    for c in chunk_offs:
                        ob[tok, pl.ds(c, 16)] = t2_vmem[xr, pl.ds(c, 16)]

            @plsc.parallel_loop(0, L // 16, unroll=2)
            def _(g):
                do_group(g * 16)

            if L % 16:
                do_group(L - 16)

        @pl.loop(0, rw // 16)
        def _(blk):
            pltpu.sync_copy(x_hbm.at[pl.ds(row0 + blk * 16, 16)], ibuf)

            @pl.loop(0, 8)
            def _(it):
                for phase in range(2):
                    s_local = it * 2 + phase
                    s = blk * 16 + s_local

                    @pl.when(s >= 2)
                    def _():
                        pltpu.make_async_copy(
                            obuf.at[phase],
                            o_hbm.at[pl.ds((row0 + s) * L, L)],
                            sems[phase]).wait()

                    compute(s_local, phase)
                    pltpu.async_copy(obuf.at[phase],
                                     o_hbm.at[pl.ds((row0 + s) * L, L)],
                                     sems[phase])

        pltpu.make_async_copy(obuf.at[0],
                              o_hbm.at[pl.ds((row0 + rw - 2) * L, L)],
                              sem0).wait()
        pltpu.make_async_copy(obuf.at[1],
                              o_hbm.at[pl.ds((row0 + rw - 1) * L, L)],
                              sem1).wait()

    return kern(table2, x)


def kernel(x, table, W1, b1, W2, b2):
    B, L = x.shape
    table2 = _compute_table2(table, W1, b1, W2, b2)
    out = _sc_gather(table2, x.astype(jnp.int32))
    return out.reshape(B, L, D_FEAT_)
